# Initial kernel scaffold; baseline (speedup 1.0000x reference)
import jax
import jax.numpy as jnp
from jax.experimental import pallas as pl


def kernel(x, edge_index, batch, W1, b1, W2, b2, node_W, node_b, graph_W, graph_b):
    raise NotImplementedError("write your pallas kernel here")


# speedup vs baseline: 11.0975x; 11.0975x over previous
import functools
import math

import jax
import jax.numpy as jnp
from jax import lax
from jax.experimental import pallas as pl
from jax.experimental.pallas import tpu as pltpu
from jax.experimental.pallas import tpu_sc as plsc

_NCORE = 2
_NSUB = 16
_M = 128
_CHUNK = 4


@functools.lru_cache(maxsize=None)
def _build_sc(E, N, H):
    ntiles = _NCORE * _NSUB
    rows_total = -(-E // _M)
    rows_per_tile = -(-(-(-rows_total // ntiles)) // _CHUNK) * _CHUNK
    totrows = ntiles * rows_per_tile
    epad = totrows * _M
    nchunk = rows_per_tile // _CHUNK
    nrow = -(-(N + 1) // 128) * 128
    init_rows = nrow // _NSUB

    mesh = plsc.VectorSubcoreMesh(core_axis_name="c", subcore_axis_name="s")
    sc_params = pltpu.CompilerParams(use_tc_tiling_on_sc=False)

    @functools.partial(
        pl.kernel,
        out_type=jax.ShapeDtypeStruct((_NCORE, nrow, H), jnp.float32),
        mesh=mesh,
        compiler_params=sc_params,
        scratch_types=[
            pltpu.VMEM((_CHUNK, _M), jnp.int32),
            pltpu.VMEM((_CHUNK, _M), jnp.int32),
            pltpu.VMEM((_CHUNK * _M, H), jnp.float32),
            pltpu.VMEM_SHARED((nrow, H), jnp.float32),
            pltpu.SemaphoreType.DMA,
        ],
    )
    def conv(g_hbm, src_hbm, dst_hbm, z_hbm, out_hbm, sidx, didx, buf, acc, sem):
        c = lax.axis_index("c")
        s = lax.axis_index("s")
        t = c * _NSUB + s
        ib = s * init_rows
        pltpu.sync_copy(z_hbm.at[pl.ds(ib, init_rows)], acc.at[pl.ds(ib, init_rows)])
        plsc.subcore_barrier()

        @pl.loop(0, nchunk)
        def _(k):
            row0 = t * rows_per_tile + k * _CHUNK
            pltpu.sync_copy(src_hbm.at[pl.ds(row0, _CHUNK)], sidx)
            pltpu.sync_copy(dst_hbm.at[pl.ds(row0, _CHUNK)], didx)
            cps = [
                pltpu.async_copy(g_hbm.at[sidx.at[j]], buf.at[pl.ds(j * _M, _M)], sem)
                for j in range(_CHUNK)
            ]
            for cp in cps:
                cp.wait()
            for j in range(_CHUNK):
                pltpu.sync_copy(buf.at[pl.ds(j * _M, _M)], acc.at[didx.at[j]], add=True)

        plsc.subcore_barrier()
        pltpu.sync_copy(acc.at[pl.ds(ib, init_rows)], out_hbm.at[c, pl.ds(ib, init_rows)])

    @functools.partial(
        pl.kernel,
        out_type=jax.ShapeDtypeStruct((_NCORE, nrow, 16), jnp.float32),
        mesh=mesh,
        compiler_params=sc_params,
        scratch_types=[
            pltpu.VMEM((_CHUNK, _M), jnp.int32),
            pltpu.VMEM((_M, 16), jnp.float32),
            pltpu.VMEM_SHARED((nrow, 16), jnp.float32),
        ],
    )
    def deg(dst_hbm, z_hbm, ones_hbm, out_hbm, didx, ones, dacc):
        c = lax.axis_index("c")
        s = lax.axis_index("s")
        t = c * _NSUB + s
        pltpu.sync_copy(ones_hbm, ones)
        ib = s * init_rows
        pltpu.sync_copy(z_hbm.at[pl.ds(ib, init_rows)], dacc.at[pl.ds(ib, init_rows)])
        plsc.subcore_barrier()

        @pl.loop(0, nchunk)
        def _(k):
            row0 = t * rows_per_tile + k * _CHUNK
            pltpu.sync_copy(dst_hbm.at[pl.ds(row0, _CHUNK)], didx)
            for j in range(_CHUNK):
                pltpu.sync_copy(ones, dacc.at[didx.at[j]], add=True)

        plsc.subcore_barrier()
        pltpu.sync_copy(dacc.at[pl.ds(ib, init_rows)], out_hbm.at[c, pl.ds(ib, init_rows)])

    return conv, deg, (epad, totrows, nrow)


@functools.lru_cache(maxsize=None)
def _build_tc(N, D, H, G):
    BN = 1000 if N % 1000 == 0 else N
    NB = N // BN
    H2 = 2 * H

    def stageb_body(x_ref, deg_ref, w_ref, g_ref, dis_ref):
        d = deg_ref[0] + deg_ref[1]
        dis = lax.rsqrt(d[:, 0:1] + 1.0)
        h = jnp.dot(x_ref[...], w_ref[...], preferred_element_type=jnp.float32)
        g_ref[...] = h * dis
        dis_ref[...] = dis

    stageb = pl.pallas_call(
        stageb_body,
        grid=(NB,),
        in_specs=[
            pl.BlockSpec((BN, D), lambda i: (i, 0)),
            pl.BlockSpec((_NCORE, BN, 16), lambda i: (0, i, 0)),
            pl.BlockSpec((D, H), lambda i: (0, 0)),
        ],
        out_specs=[
            pl.BlockSpec((BN, H), lambda i: (i, 0)),
            pl.BlockSpec((BN, 1), lambda i: (i, 0)),
        ],
        out_shape=[
            jax.ShapeDtypeStruct((N, H), jnp.float32),
            jax.ShapeDtypeStruct((N, 1), jnp.float32),
        ],
    )

    def staged_body(a_ref, g1_ref, dis_ref, b_ref, w_ref, h1_ref, g2_ref):
        dis = dis_ref[...]
        acc = a_ref[0] + a_ref[1] + g1_ref[...]
        h1 = jnp.maximum(acc * dis + b_ref[...], 0.0)
        h1_ref[...] = h1
        g2_ref[...] = jnp.dot(h1, w_ref[...], preferred_element_type=jnp.float32) * dis

    staged = pl.pallas_call(
        staged_body,
        grid=(NB,),
        in_specs=[
            pl.BlockSpec((_NCORE, BN, H), lambda i: (0, i, 0)),
            pl.BlockSpec((BN, H), lambda i: (i, 0)),
            pl.BlockSpec((BN, 1), lambda i: (i, 0)),
            pl.BlockSpec((1, H), lambda i: (0, 0)),
            pl.BlockSpec((H, H), lambda i: (0, 0)),
        ],
        out_specs=[
            pl.BlockSpec((BN, H), lambda i: (i, 0)),
            pl.BlockSpec((BN, H), lambda i: (i, 0)),
        ],
        out_shape=[
            jax.ShapeDtypeStruct((N, H), jnp.float32),
            jax.ShapeDtypeStruct((N, H), jnp.float32),
        ],
    )

    def stagef_body(a_ref, g2_ref, dis_ref, b_ref, h1_ref, batch_ref, nw_ref,
                    nb_ref, gw_ref, gb_ref, nl_ref, gl_ref, sums_ref, cnt_ref,
                    mx_ref):
        i = pl.program_id(0)
        dis = dis_ref[...]
        acc = a_ref[0] + a_ref[1] + g2_ref[...]
        h2 = jnp.maximum(acc * dis + b_ref[...], 0.0)
        h = jnp.concatenate([h1_ref[...], h2], axis=1)
        nl_ref[...] = (
            jnp.dot(h, nw_ref[...], preferred_element_type=jnp.float32) + nb_ref[...]
        )

        @pl.when(i == 0)
        def _():
            sums_ref[...] = jnp.zeros_like(sums_ref)
            cnt_ref[...] = jnp.zeros_like(cnt_ref)
            mx_ref[...] = jnp.full_like(mx_ref, -jnp.inf)

        brow = batch_ref[0]
        iota_g = lax.broadcasted_iota(jnp.int32, (G, BN), 0)
        maskf = (brow == iota_g).astype(jnp.float32)
        sums_ref[...] += jnp.dot(maskf, h, preferred_element_type=jnp.float32)
        cnt_ref[...] += jnp.sum(maskf, axis=1, keepdims=True)
        bcol = brow.reshape(BN, 1)
        for g in range(G):
            m = jnp.max(jnp.where(bcol == g, h, -jnp.inf), axis=0, keepdims=True)
            mx_ref[pl.ds(g, 1), :] = jnp.maximum(mx_ref[pl.ds(g, 1), :], m)

        cnt = cnt_ref[...]
        mean = sums_ref[...] / jnp.maximum(cnt, 1.0)
        mxp = jnp.where(cnt > 0, mx_ref[...], 0.0)
        pooled = jnp.concatenate([mean, mxp], axis=1)
        gl_ref[...] = (
            jnp.dot(pooled, gw_ref[...], preferred_element_type=jnp.float32)
            + gb_ref[...]
        )

    stagef = pl.pallas_call(
        stagef_body,
        grid=(NB,),
        in_specs=[
            pl.BlockSpec((_NCORE, BN, H), lambda i: (0, i, 0)),
            pl.BlockSpec((BN, H), lambda i: (i, 0)),
            pl.BlockSpec((BN, 1), lambda i: (i, 0)),
            pl.BlockSpec((1, H), lambda i: (0, 0)),
            pl.BlockSpec((BN, H), lambda i: (i, 0)),
            pl.BlockSpec((1, 1, BN), lambda i: (i, 0, 0)),
            pl.BlockSpec((H2, 1), lambda i: (0, 0)),
            pl.BlockSpec((1, 1), lambda i: (0, 0)),
            pl.BlockSpec((2 * H2, 1), lambda i: (0, 0)),
            pl.BlockSpec((1, 1), lambda i: (0, 0)),
        ],
        out_specs=[
            pl.BlockSpec((BN, 1), lambda i: (i, 0)),
            pl.BlockSpec((G, 1), lambda i: (0, 0)),
        ],
        out_shape=[
            jax.ShapeDtypeStruct((N, 1), jnp.float32),
            jax.ShapeDtypeStruct((G, 1), jnp.float32),
        ],
        scratch_shapes=[
            pltpu.VMEM((G, H2), jnp.float32),
            pltpu.VMEM((G, 1), jnp.float32),
            pltpu.VMEM((G, H2), jnp.float32),
        ],
        compiler_params=pltpu.CompilerParams(
            dimension_semantics=("arbitrary",),
        ),
    )

    return stageb, staged, stagef, (BN, NB)


def kernel(x, edge_index, batch, W1, b1, W2, b2, node_W, node_b, graph_W, graph_b):
    N, D = x.shape
    H = W1.shape[0]
    G = 16
    E = edge_index.shape[1]

    conv_sc, deg_sc, (epad, totrows, nrow) = _build_sc(E, N, H)
    stageb, staged, stagef, (BN, NB) = _build_tc(N, D, H, G)

    pad = epad - E
    src = jnp.concatenate(
        [edge_index[0], jnp.zeros((pad,), jnp.int32)]).reshape(totrows, _M)
    dst = jnp.concatenate(
        [edge_index[1], jnp.full((pad,), N, jnp.int32)]).reshape(totrows, _M)
    z_h = jnp.zeros((nrow, H), jnp.float32)
    z_16 = jnp.zeros((nrow, 16), jnp.float32)
    ones = jnp.ones((_M, 16), jnp.float32)

    degs = deg_sc(dst, z_16, ones)[:, :N, :]
    g1, dis = stageb(x, degs, W1.T)
    a1 = conv_sc(g1, src, dst, z_h)[:, :N, :]
    h1, g2 = staged(a1, g1, dis, b1.reshape(1, H), W2.T)
    a2 = conv_sc(g2, src, dst, z_h)[:, :N, :]
    nl, gl = stagef(
        a2, g2, dis, b2.reshape(1, H), h1,
        batch.reshape(NB, 1, BN),
        node_W.reshape(2 * H, 1), node_b.reshape(1, 1),
        graph_W.reshape(4 * H, 1), graph_b.reshape(1, 1),
    )
    return (gl.reshape(-1), nl.reshape(-1))


# speedup vs baseline: 12.4109x; 1.1183x over previous
import functools
import math

import jax
import jax.numpy as jnp
from jax import lax
from jax.experimental import pallas as pl
from jax.experimental.pallas import tpu as pltpu
from jax.experimental.pallas import tpu_sc as plsc

_NCORE = 2
_NSUB = 16
_M = 128
_CHUNK = 4


@functools.lru_cache(maxsize=None)
def _build_sc(E, N, H):
    ntiles = _NCORE * _NSUB
    rows_total = -(-E // _M)
    rows_per_tile = -(-(-(-rows_total // ntiles)) // _CHUNK) * _CHUNK
    totrows = ntiles * rows_per_tile
    epad = totrows * _M
    nchunk = rows_per_tile // _CHUNK
    nrow = -(-(N + 1) // 128) * 128
    init_rows = nrow // _NSUB

    mesh = plsc.VectorSubcoreMesh(core_axis_name="c", subcore_axis_name="s")
    sc_params = pltpu.CompilerParams(use_tc_tiling_on_sc=False)

    ch = 2
    nch = rows_per_tile // ch

    @functools.partial(
        pl.kernel,
        out_type=jax.ShapeDtypeStruct((_NCORE, nrow, H), jnp.float32),
        mesh=mesh,
        compiler_params=sc_params,
        scratch_types=[
            pltpu.VMEM((rows_per_tile, _M), jnp.int32),
            pltpu.VMEM((rows_per_tile, _M), jnp.int32),
            pltpu.VMEM((ch * _M, H), jnp.float32),
            pltpu.VMEM((ch * _M, H), jnp.float32),
            pltpu.VMEM_SHARED((nrow, H), jnp.float32),
            pltpu.SemaphoreType.DMA,
            pltpu.SemaphoreType.DMA,
        ],
    )
    def conv(g_hbm, src_hbm, dst_hbm, z_hbm, out_hbm, sidx, didx, buf0, buf1,
             acc, sem0, sem1):
        c = lax.axis_index("c")
        s = lax.axis_index("s")
        t = c * _NSUB + s
        ib = s * init_rows
        row_t = t * rows_per_tile
        pltpu.sync_copy(src_hbm.at[pl.ds(row_t, rows_per_tile)], sidx)
        pltpu.sync_copy(dst_hbm.at[pl.ds(row_t, rows_per_tile)], didx)
        pltpu.sync_copy(z_hbm.at[pl.ds(ib, init_rows)], acc.at[pl.ds(ib, init_rows)])
        plsc.subcore_barrier()

        bufs = (buf0, buf1)
        sems = (sem0, sem1)

        def fire(k, p):
            for j in range(ch):
                pltpu.async_copy(g_hbm.at[sidx.at[k * ch + j]],
                                 bufs[p].at[pl.ds(j * _M, _M)], sems[p])

        def wait(k, p):
            for j in range(ch):
                pltpu.make_async_copy(g_hbm.at[sidx.at[k * ch + j]],
                                      bufs[p].at[pl.ds(j * _M, _M)],
                                      sems[p]).wait()

        def scat(k, p):
            for j in range(ch):
                pltpu.sync_copy(bufs[p].at[pl.ds(j * _M, _M)],
                                acc.at[didx.at[k * ch + j]], add=True)

        fire(0, 0)

        @pl.loop(0, nch - 2, step=2)
        def _(k):
            fire(k + 1, 1)
            wait(k, 0)
            scat(k, 0)
            fire(k + 2, 0)
            wait(k + 1, 1)
            scat(k + 1, 1)

        fire(nch - 1, 1)
        wait(nch - 2, 0)
        scat(nch - 2, 0)
        wait(nch - 1, 1)
        scat(nch - 1, 1)

        plsc.subcore_barrier()
        pltpu.sync_copy(acc.at[pl.ds(ib, init_rows)], out_hbm.at[c, pl.ds(ib, init_rows)])

    @functools.partial(
        pl.kernel,
        out_type=jax.ShapeDtypeStruct((_NCORE, nrow, 16), jnp.float32),
        mesh=mesh,
        compiler_params=sc_params,
        scratch_types=[
            pltpu.VMEM((_CHUNK, _M), jnp.int32),
            pltpu.VMEM((_M, 16), jnp.float32),
            pltpu.VMEM_SHARED((nrow, 16), jnp.float32),
        ],
    )
    def deg(dst_hbm, z_hbm, ones_hbm, out_hbm, didx, ones, dacc):
        c = lax.axis_index("c")
        s = lax.axis_index("s")
        t = c * _NSUB + s
        pltpu.sync_copy(ones_hbm, ones)
        ib = s * init_rows
        pltpu.sync_copy(z_hbm.at[pl.ds(ib, init_rows)], dacc.at[pl.ds(ib, init_rows)])
        plsc.subcore_barrier()

        @pl.loop(0, nchunk)
        def _(k):
            row0 = t * rows_per_tile + k * _CHUNK
            pltpu.sync_copy(dst_hbm.at[pl.ds(row0, _CHUNK)], didx)
            for j in range(_CHUNK):
                pltpu.sync_copy(ones, dacc.at[didx.at[j]], add=True)

        plsc.subcore_barrier()
        pltpu.sync_copy(dacc.at[pl.ds(ib, init_rows)], out_hbm.at[c, pl.ds(ib, init_rows)])

    return conv, deg, (epad, totrows, nrow)


@functools.lru_cache(maxsize=None)
def _build_tc(N, D, H, G):
    BN = 1000 if N % 1000 == 0 else N
    NB = N // BN
    H2 = 2 * H

    def stageb_body(x_ref, deg_ref, w_ref, g_ref, dis_ref):
        d = deg_ref[0] + deg_ref[1]
        dis = lax.rsqrt(d[:, 0:1] + 1.0)
        h = jnp.dot(x_ref[...], w_ref[...], preferred_element_type=jnp.float32)
        g_ref[...] = h * dis
        dis_ref[...] = dis

    stageb = pl.pallas_call(
        stageb_body,
        grid=(NB,),
        in_specs=[
            pl.BlockSpec((BN, D), lambda i: (i, 0)),
            pl.BlockSpec((_NCORE, BN, 16), lambda i: (0, i, 0)),
            pl.BlockSpec((D, H), lambda i: (0, 0)),
        ],
        out_specs=[
            pl.BlockSpec((BN, H), lambda i: (i, 0)),
            pl.BlockSpec((BN, 1), lambda i: (i, 0)),
        ],
        out_shape=[
            jax.ShapeDtypeStruct((N, H), jnp.float32),
            jax.ShapeDtypeStruct((N, 1), jnp.float32),
        ],
    )

    def staged_body(a_ref, g1_ref, dis_ref, b_ref, w_ref, h1_ref, g2_ref):
        dis = dis_ref[...]
        acc = a_ref[0] + a_ref[1] + g1_ref[...]
        h1 = jnp.maximum(acc * dis + b_ref[...], 0.0)
        h1_ref[...] = h1
        g2_ref[...] = jnp.dot(h1, w_ref[...], preferred_element_type=jnp.float32) * dis

    staged = pl.pallas_call(
        staged_body,
        grid=(NB,),
        in_specs=[
            pl.BlockSpec((_NCORE, BN, H), lambda i: (0, i, 0)),
            pl.BlockSpec((BN, H), lambda i: (i, 0)),
            pl.BlockSpec((BN, 1), lambda i: (i, 0)),
            pl.BlockSpec((1, H), lambda i: (0, 0)),
            pl.BlockSpec((H, H), lambda i: (0, 0)),
        ],
        out_specs=[
            pl.BlockSpec((BN, H), lambda i: (i, 0)),
            pl.BlockSpec((BN, H), lambda i: (i, 0)),
        ],
        out_shape=[
            jax.ShapeDtypeStruct((N, H), jnp.float32),
            jax.ShapeDtypeStruct((N, H), jnp.float32),
        ],
    )

    def stagef_body(a_ref, g2_ref, dis_ref, b_ref, h1_ref, batch_ref, nw_ref,
                    nb_ref, gw_ref, gb_ref, nl_ref, gl_ref, sums_ref, cnt_ref,
                    mx_ref):
        i = pl.program_id(0)
        dis = dis_ref[...]
        acc = a_ref[0] + a_ref[1] + g2_ref[...]
        h2 = jnp.maximum(acc * dis + b_ref[...], 0.0)
        h = jnp.concatenate([h1_ref[...], h2], axis=1)
        nl_ref[...] = (
            jnp.dot(h, nw_ref[...], preferred_element_type=jnp.float32) + nb_ref[...]
        )

        @pl.when(i == 0)
        def _():
            sums_ref[...] = jnp.zeros_like(sums_ref)
            cnt_ref[...] = jnp.zeros_like(cnt_ref)
            mx_ref[...] = jnp.full_like(mx_ref, -jnp.inf)

        brow = batch_ref[0]
        iota_g = lax.broadcasted_iota(jnp.int32, (G, BN), 0)
        maskf = (brow == iota_g).astype(jnp.float32)
        sums_ref[...] += jnp.dot(maskf, h, preferred_element_type=jnp.float32)
        cnt_ref[...] += jnp.sum(maskf, axis=1, keepdims=True)
        bcol = brow.reshape(BN, 1)
        for g in range(G):
            m = jnp.max(jnp.where(bcol == g, h, -jnp.inf), axis=0, keepdims=True)
            mx_ref[pl.ds(g, 1), :] = jnp.maximum(mx_ref[pl.ds(g, 1), :], m)

        cnt = cnt_ref[...]
        mean = sums_ref[...] / jnp.maximum(cnt, 1.0)
        mxp = jnp.where(cnt > 0, mx_ref[...], 0.0)
        pooled = jnp.concatenate([mean, mxp], axis=1)
        gl_ref[...] = (
            jnp.dot(pooled, gw_ref[...], preferred_element_type=jnp.float32)
            + gb_ref[...]
        )

    stagef = pl.pallas_call(
        stagef_body,
        grid=(NB,),
        in_specs=[
            pl.BlockSpec((_NCORE, BN, H), lambda i: (0, i, 0)),
            pl.BlockSpec((BN, H), lambda i: (i, 0)),
            pl.BlockSpec((BN, 1), lambda i: (i, 0)),
            pl.BlockSpec((1, H), lambda i: (0, 0)),
            pl.BlockSpec((BN, H), lambda i: (i, 0)),
            pl.BlockSpec((1, 1, BN), lambda i: (i, 0, 0)),
            pl.BlockSpec((H2, 1), lambda i: (0, 0)),
            pl.BlockSpec((1, 1), lambda i: (0, 0)),
            pl.BlockSpec((2 * H2, 1), lambda i: (0, 0)),
            pl.BlockSpec((1, 1), lambda i: (0, 0)),
        ],
        out_specs=[
            pl.BlockSpec((BN, 1), lambda i: (i, 0)),
            pl.BlockSpec((G, 1), lambda i: (0, 0)),
        ],
        out_shape=[
            jax.ShapeDtypeStruct((N, 1), jnp.float32),
            jax.ShapeDtypeStruct((G, 1), jnp.float32),
        ],
        scratch_shapes=[
            pltpu.VMEM((G, H2), jnp.float32),
            pltpu.VMEM((G, 1), jnp.float32),
            pltpu.VMEM((G, H2), jnp.float32),
        ],
        compiler_params=pltpu.CompilerParams(
            dimension_semantics=("arbitrary",),
        ),
    )

    return stageb, staged, stagef, (BN, NB)


def kernel(x, edge_index, batch, W1, b1, W2, b2, node_W, node_b, graph_W, graph_b):
    N, D = x.shape
    H = W1.shape[0]
    G = 16
    E = edge_index.shape[1]

    conv_sc, deg_sc, (epad, totrows, nrow) = _build_sc(E, N, H)
    stageb, staged, stagef, (BN, NB) = _build_tc(N, D, H, G)

    pad = epad - E
    src = jnp.concatenate(
        [edge_index[0], jnp.zeros((pad,), jnp.int32)]).reshape(totrows, _M)
    dst = jnp.concatenate(
        [edge_index[1], jnp.full((pad,), N, jnp.int32)]).reshape(totrows, _M)
    z_h = jnp.zeros((nrow, H), jnp.float32)
    z_16 = jnp.zeros((nrow, 16), jnp.float32)
    ones = jnp.ones((_M, 16), jnp.float32)

    degs = deg_sc(dst, z_16, ones)[:, :N, :]
    g1, dis = stageb(x, degs, W1.T)
    a1 = conv_sc(g1, src, dst, z_h)[:, :N, :]
    h1, g2 = staged(a1, g1, dis, b1.reshape(1, H), W2.T)
    a2 = conv_sc(g2, src, dst, z_h)[:, :N, :]
    nl, gl = stagef(
        a2, g2, dis, b2.reshape(1, H), h1,
        batch.reshape(NB, 1, BN),
        node_W.reshape(2 * H, 1), node_b.reshape(1, 1),
        graph_W.reshape(4 * H, 1), graph_b.reshape(1, 1),
    )
    return (gl.reshape(-1), nl.reshape(-1))


# speedup vs baseline: 29.4464x; 2.3726x over previous
import functools
import math

import jax
import jax.numpy as jnp
from jax import lax
from jax.experimental import pallas as pl
from jax.experimental.pallas import tpu as pltpu
from jax.experimental.pallas import tpu_sc as plsc

_NCORE = 2
_NSUB = 16
_M = 128
_CHUNK = 4


@functools.lru_cache(maxsize=None)
def _build_sc(E, N, H):
    ntiles = _NCORE * _NSUB
    rows_total = -(-E // _M)
    rows_per_tile = -(-(-(-rows_total // ntiles)) // _CHUNK) * _CHUNK
    totrows = ntiles * rows_per_tile
    epad = totrows * _M
    nchunk = rows_per_tile // _CHUNK
    nrow = -(-(N + 1) // 128) * 128
    init_rows = nrow // _NSUB

    mesh = plsc.VectorSubcoreMesh(core_axis_name="c", subcore_axis_name="s")
    sc_params = pltpu.CompilerParams(use_tc_tiling_on_sc=False)

    ch = 2
    nch = rows_per_tile // ch

    @functools.partial(
        pl.kernel,
        out_type=jax.ShapeDtypeStruct((_NCORE, nrow, H), jnp.float32),
        mesh=mesh,
        compiler_params=sc_params,
        scratch_types=[
            pltpu.VMEM((rows_per_tile, _M), jnp.int32),
            pltpu.VMEM((rows_per_tile, _M), jnp.int32),
            pltpu.VMEM((ch * _M, H), jnp.float32),
            pltpu.VMEM((ch * _M, H), jnp.float32),
            pltpu.VMEM_SHARED((nrow, H), jnp.float32),
            pltpu.SemaphoreType.DMA,
            pltpu.SemaphoreType.DMA,
        ],
    )
    def conv(g_hbm, src_hbm, dst_hbm, z_hbm, out_hbm, sidx, didx, buf0, buf1,
             acc, sem0, sem1):
        c = lax.axis_index("c")
        s = lax.axis_index("s")
        t = c * _NSUB + s
        ib = s * init_rows
        row_t = t * rows_per_tile
        pltpu.sync_copy(src_hbm.at[pl.ds(row_t, rows_per_tile)], sidx)
        pltpu.sync_copy(dst_hbm.at[pl.ds(row_t, rows_per_tile)], didx)
        pltpu.sync_copy(z_hbm.at[pl.ds(ib, init_rows)], acc.at[pl.ds(ib, init_rows)])
        plsc.subcore_barrier()

        bufs = (buf0, buf1)
        sems = (sem0, sem1)

        def fire(k, p):
            for j in range(ch):
                pltpu.async_copy(g_hbm.at[sidx.at[k * ch + j]],
                                 bufs[p].at[pl.ds(j * _M, _M)], sems[p])

        def wait(k, p):
            for j in range(ch):
                pltpu.make_async_copy(g_hbm.at[sidx.at[k * ch + j]],
                                      bufs[p].at[pl.ds(j * _M, _M)],
                                      sems[p]).wait()

        def scat(k, p):
            for j in range(ch):
                pltpu.sync_copy(bufs[p].at[pl.ds(j * _M, _M)],
                                acc.at[didx.at[k * ch + j]], add=True)

        fire(0, 0)

        @pl.loop(0, nch - 2, step=2)
        def _(k):
            fire(k + 1, 1)
            wait(k, 0)
            scat(k, 0)
            fire(k + 2, 0)
            wait(k + 1, 1)
            scat(k + 1, 1)

        fire(nch - 1, 1)
        wait(nch - 2, 0)
        scat(nch - 2, 0)
        wait(nch - 1, 1)
        scat(nch - 1, 1)

        plsc.subcore_barrier()
        pltpu.sync_copy(acc.at[pl.ds(ib, init_rows)], out_hbm.at[c, pl.ds(ib, init_rows)])

    @functools.partial(
        pl.kernel,
        out_type=jax.ShapeDtypeStruct((_NCORE, nrow, 16), jnp.float32),
        mesh=mesh,
        compiler_params=sc_params,
        scratch_types=[
            pltpu.VMEM((_CHUNK, _M), jnp.int32),
            pltpu.VMEM((_M, 16), jnp.float32),
            pltpu.VMEM_SHARED((nrow, 16), jnp.float32),
        ],
    )
    def deg(dst_hbm, z_hbm, ones_hbm, out_hbm, didx, ones, dacc):
        c = lax.axis_index("c")
        s = lax.axis_index("s")
        t = c * _NSUB + s
        pltpu.sync_copy(ones_hbm, ones)
        ib = s * init_rows
        pltpu.sync_copy(z_hbm.at[pl.ds(ib, init_rows)], dacc.at[pl.ds(ib, init_rows)])
        plsc.subcore_barrier()

        @pl.loop(0, nchunk)
        def _(k):
            row0 = t * rows_per_tile + k * _CHUNK
            pltpu.sync_copy(dst_hbm.at[pl.ds(row0, _CHUNK)], didx)
            for j in range(_CHUNK):
                pltpu.sync_copy(ones, dacc.at[didx.at[j]], add=True)

        plsc.subcore_barrier()
        pltpu.sync_copy(dacc.at[pl.ds(ib, init_rows)], out_hbm.at[c, pl.ds(ib, init_rows)])

    return conv, deg, (epad, totrows, nrow)


@functools.lru_cache(maxsize=None)
def _build_tc(N, D, H, G):
    BN = 1000 if N % 1000 == 0 else N
    NB = N // BN
    H2 = 2 * H

    def stageb_body(x_ref, deg_ref, w_ref, g_ref, dis_ref):
        d = deg_ref[0] + deg_ref[1]
        dis = lax.rsqrt(d[:, 0:1] + 1.0)
        h = jnp.dot(x_ref[...], w_ref[...], preferred_element_type=jnp.float32)
        g_ref[...] = h * dis
        dis_ref[...] = dis

    stageb = pl.pallas_call(
        stageb_body,
        grid=(NB,),
        in_specs=[
            pl.BlockSpec((BN, D), lambda i: (i, 0)),
            pl.BlockSpec((_NCORE, BN, 16), lambda i: (0, i, 0)),
            pl.BlockSpec((D, H), lambda i: (0, 0)),
        ],
        out_specs=[
            pl.BlockSpec((BN, H), lambda i: (i, 0)),
            pl.BlockSpec((BN, 1), lambda i: (i, 0)),
        ],
        out_shape=[
            jax.ShapeDtypeStruct((N, H), jnp.float32),
            jax.ShapeDtypeStruct((N, 1), jnp.float32),
        ],
    )

    def staged_body(a_ref, g1_ref, dis_ref, b_ref, w_ref, h1_ref, g2_ref):
        dis = dis_ref[...]
        acc = a_ref[0] + a_ref[1] + g1_ref[...]
        h1 = jnp.maximum(acc * dis + b_ref[...], 0.0)
        h1_ref[...] = h1
        g2_ref[...] = jnp.dot(h1, w_ref[...], preferred_element_type=jnp.float32) * dis

    staged = pl.pallas_call(
        staged_body,
        grid=(NB,),
        in_specs=[
            pl.BlockSpec((_NCORE, BN, H), lambda i: (0, i, 0)),
            pl.BlockSpec((BN, H), lambda i: (i, 0)),
            pl.BlockSpec((BN, 1), lambda i: (i, 0)),
            pl.BlockSpec((1, H), lambda i: (0, 0)),
            pl.BlockSpec((H, H), lambda i: (0, 0)),
        ],
        out_specs=[
            pl.BlockSpec((BN, H), lambda i: (i, 0)),
            pl.BlockSpec((BN, H), lambda i: (i, 0)),
        ],
        out_shape=[
            jax.ShapeDtypeStruct((N, H), jnp.float32),
            jax.ShapeDtypeStruct((N, H), jnp.float32),
        ],
    )

    def stagef_body(a_ref, g2_ref, dis_ref, b_ref, h1_ref, batch_ref, nw_ref,
                    nb_ref, gw_ref, gb_ref, nl_ref, gl_ref, sums_ref, cnt_ref,
                    mx_ref):
        i = pl.program_id(0)
        dis = dis_ref[...]
        acc = a_ref[0] + a_ref[1] + g2_ref[...]
        h2 = jnp.maximum(acc * dis + b_ref[...], 0.0)
        h = jnp.concatenate([h1_ref[...], h2], axis=1)
        nl_ref[...] = (
            jnp.dot(h, nw_ref[...], preferred_element_type=jnp.float32) + nb_ref[...]
        )

        @pl.when(i == 0)
        def _():
            sums_ref[...] = jnp.zeros_like(sums_ref)
            cnt_ref[...] = jnp.zeros_like(cnt_ref)
            mx_ref[...] = jnp.full_like(mx_ref, -jnp.inf)

        brow = batch_ref[0]
        iota_g = lax.broadcasted_iota(jnp.int32, (G, BN), 0)
        maskf = (brow == iota_g).astype(jnp.float32)
        sums_ref[...] += jnp.dot(maskf, h, preferred_element_type=jnp.float32)
        cnt_ref[...] += jnp.sum(maskf, axis=1, keepdims=True)
        bcol = brow.reshape(BN, 1)
        for g in range(G):
            m = jnp.max(jnp.where(bcol == g, h, -jnp.inf), axis=0, keepdims=True)
            mx_ref[pl.ds(g, 1), :] = jnp.maximum(mx_ref[pl.ds(g, 1), :], m)

        cnt = cnt_ref[...]
        mean = sums_ref[...] / jnp.maximum(cnt, 1.0)
        mxp = jnp.where(cnt > 0, mx_ref[...], 0.0)
        pooled = jnp.concatenate([mean, mxp], axis=1)
        gl_ref[...] = (
            jnp.dot(pooled, gw_ref[...], preferred_element_type=jnp.float32)
            + gb_ref[...]
        )

    stagef = pl.pallas_call(
        stagef_body,
        grid=(NB,),
        in_specs=[
            pl.BlockSpec((_NCORE, BN, H), lambda i: (0, i, 0)),
            pl.BlockSpec((BN, H), lambda i: (i, 0)),
            pl.BlockSpec((BN, 1), lambda i: (i, 0)),
            pl.BlockSpec((1, H), lambda i: (0, 0)),
            pl.BlockSpec((BN, H), lambda i: (i, 0)),
            pl.BlockSpec((1, 1, BN), lambda i: (i, 0, 0)),
            pl.BlockSpec((H2, 1), lambda i: (0, 0)),
            pl.BlockSpec((1, 1), lambda i: (0, 0)),
            pl.BlockSpec((2 * H2, 1), lambda i: (0, 0)),
            pl.BlockSpec((1, 1), lambda i: (0, 0)),
        ],
        out_specs=[
            pl.BlockSpec((BN, 1), lambda i: (i, 0)),
            pl.BlockSpec((G, 1), lambda i: (0, 0)),
        ],
        out_shape=[
            jax.ShapeDtypeStruct((N, 1), jnp.float32),
            jax.ShapeDtypeStruct((G, 1), jnp.float32),
        ],
        scratch_shapes=[
            pltpu.VMEM((G, H2), jnp.float32),
            pltpu.VMEM((G, 1), jnp.float32),
            pltpu.VMEM((G, H2), jnp.float32),
        ],
        compiler_params=pltpu.CompilerParams(
            dimension_semantics=("arbitrary",),
        ),
    )

    return stageb, staged, stagef, (BN, NB)


def kernel(x, edge_index, batch, W1, b1, W2, b2, node_W, node_b, graph_W, graph_b):
    N, D = x.shape
    H = W1.shape[0]
    G = 16
    E = edge_index.shape[1]

    conv_sc, deg_sc, (epad, totrows, nrow) = _build_sc(E, N, H)
    stageb, staged, stagef, (BN, NB) = _build_tc(N, D, H, G)

    pad = epad - E
    pad_i = jnp.arange(pad, dtype=jnp.int32)
    src = jnp.concatenate(
        [edge_index[0], pad_i % N]).reshape(totrows, _M)
    dst = jnp.concatenate(
        [edge_index[1], N + pad_i % (nrow - N)]).reshape(totrows, _M)
    z_h = jnp.zeros((nrow, H), jnp.float32)
    z_16 = jnp.zeros((nrow, 16), jnp.float32)
    ones = jnp.ones((_M, 16), jnp.float32)

    degs = deg_sc(dst, z_16, ones)[:, :N, :]
    g1, dis = stageb(x, degs, W1.T)
    a1 = conv_sc(g1, src, dst, z_h)[:, :N, :]
    h1, g2 = staged(a1, g1, dis, b1.reshape(1, H), W2.T)
    a2 = conv_sc(g2, src, dst, z_h)[:, :N, :]
    nl, gl = stagef(
        a2, g2, dis, b2.reshape(1, H), h1,
        batch.reshape(NB, 1, BN),
        node_W.reshape(2 * H, 1), node_b.reshape(1, 1),
        graph_W.reshape(4 * H, 1), graph_b.reshape(1, 1),
    )
    return (gl.reshape(-1), nl.reshape(-1))


# speedup vs baseline: 31.4135x; 1.0668x over previous
import functools
import math

import jax
import jax.numpy as jnp
from jax import lax
from jax.experimental import pallas as pl
from jax.experimental.pallas import tpu as pltpu
from jax.experimental.pallas import tpu_sc as plsc

_NCORE = 2
_NSUB = 16
_M = 128
_CHUNK = 4


@functools.lru_cache(maxsize=None)
def _build_sc(E, N, H):
    ntiles = _NCORE * _NSUB
    rows_total = -(-E // _M)
    rows_per_tile = -(-(-(-rows_total // ntiles)) // _CHUNK) * _CHUNK
    totrows = ntiles * rows_per_tile
    epad = totrows * _M
    nchunk = rows_per_tile // _CHUNK
    nrow = -(-(N + 1) // 128) * 128
    init_rows = nrow // _NSUB

    mesh = plsc.VectorSubcoreMesh(core_axis_name="c", subcore_axis_name="s")
    sc_params = pltpu.CompilerParams(use_tc_tiling_on_sc=False)

    ch = 2
    nch = rows_per_tile // ch

    @functools.partial(
        pl.kernel,
        out_type=jax.ShapeDtypeStruct((_NCORE, nrow, H), jnp.float32),
        mesh=mesh,
        compiler_params=sc_params,
        scratch_types=[
            pltpu.VMEM((rows_per_tile, _M), jnp.int32),
            pltpu.VMEM((rows_per_tile, _M), jnp.int32),
            pltpu.VMEM((ch * _M, H), jnp.float32),
            pltpu.VMEM((ch * _M, H), jnp.float32),
            pltpu.VMEM_SHARED((nrow, H), jnp.float32),
            pltpu.SemaphoreType.DMA,
            pltpu.SemaphoreType.DMA,
        ],
    )
    def conv(g_hbm, src_hbm, dst_hbm, z_hbm, out_hbm, sidx, didx, buf0, buf1,
             acc, sem0, sem1):
        c = lax.axis_index("c")
        s = lax.axis_index("s")
        t = c * _NSUB + s
        ib = s * init_rows
        row_t = t * rows_per_tile
        pltpu.sync_copy(src_hbm.at[pl.ds(row_t, rows_per_tile)], sidx)
        pltpu.sync_copy(dst_hbm.at[pl.ds(row_t, rows_per_tile)], didx)
        pltpu.sync_copy(z_hbm.at[pl.ds(ib, init_rows)], acc.at[pl.ds(ib, init_rows)])
        plsc.subcore_barrier()

        bufs = (buf0, buf1)
        sems = (sem0, sem1)

        def fire(k, p):
            for j in range(ch):
                pltpu.async_copy(g_hbm.at[sidx.at[k * ch + j]],
                                 bufs[p].at[pl.ds(j * _M, _M)], sems[p])

        def wait(k, p):
            for j in range(ch):
                pltpu.make_async_copy(g_hbm.at[sidx.at[k * ch + j]],
                                      bufs[p].at[pl.ds(j * _M, _M)],
                                      sems[p]).wait()

        def scat(k, p):
            for j in range(ch):
                pltpu.sync_copy(bufs[p].at[pl.ds(j * _M, _M)],
                                acc.at[didx.at[k * ch + j]], add=True)

        fire(0, 0)

        @pl.loop(0, nch - 2, step=2)
        def _(k):
            fire(k + 1, 1)
            wait(k, 0)
            scat(k, 0)
            fire(k + 2, 0)
            wait(k + 1, 1)
            scat(k + 1, 1)

        fire(nch - 1, 1)
        wait(nch - 2, 0)
        scat(nch - 2, 0)
        wait(nch - 1, 1)
        scat(nch - 1, 1)

        plsc.subcore_barrier()
        pltpu.sync_copy(acc.at[pl.ds(ib, init_rows)], out_hbm.at[c, pl.ds(ib, init_rows)])

    @functools.partial(
        pl.kernel,
        out_type=jax.ShapeDtypeStruct((_NCORE, nrow, 16), jnp.float32),
        mesh=mesh,
        compiler_params=sc_params,
        scratch_types=[
            pltpu.VMEM((_CHUNK, _M), jnp.int32),
            pltpu.VMEM((_M, 16), jnp.float32),
            pltpu.VMEM_SHARED((nrow, 16), jnp.float32),
        ],
    )
    def deg(dst_hbm, z_hbm, ones_hbm, out_hbm, didx, ones, dacc):
        c = lax.axis_index("c")
        s = lax.axis_index("s")
        t = c * _NSUB + s
        pltpu.sync_copy(ones_hbm, ones)
        ib = s * init_rows
        pltpu.sync_copy(z_hbm.at[pl.ds(ib, init_rows)], dacc.at[pl.ds(ib, init_rows)])
        plsc.subcore_barrier()

        @pl.loop(0, nchunk)
        def _(k):
            row0 = t * rows_per_tile + k * _CHUNK
            pltpu.sync_copy(dst_hbm.at[pl.ds(row0, _CHUNK)], didx)
            for j in range(_CHUNK):
                pltpu.sync_copy(ones, dacc.at[didx.at[j]], add=True)

        plsc.subcore_barrier()
        pltpu.sync_copy(dacc.at[pl.ds(ib, init_rows)], out_hbm.at[c, pl.ds(ib, init_rows)])

    return conv, deg, (epad, totrows, nrow)


@functools.lru_cache(maxsize=None)
def _build_tc(N, D, H, G):
    BN = 1000 if N % 1000 == 0 else N
    NB = N // BN
    H2 = 2 * H

    def stageb_body(x_ref, deg_ref, w_ref, g_ref, dis_ref):
        d = deg_ref[0] + deg_ref[1]
        dis = lax.rsqrt(d[:, 0:1] + 1.0)
        h = jnp.dot(x_ref[...], w_ref[...], preferred_element_type=jnp.float32)
        g_ref[...] = h * dis
        dis_ref[...] = dis

    stageb = pl.pallas_call(
        stageb_body,
        grid=(NB,),
        in_specs=[
            pl.BlockSpec((BN, D), lambda i: (i, 0)),
            pl.BlockSpec((_NCORE, BN, 16), lambda i: (0, i, 0)),
            pl.BlockSpec((D, H), lambda i: (0, 0)),
        ],
        out_specs=[
            pl.BlockSpec((BN, H), lambda i: (i, 0)),
            pl.BlockSpec((BN, 1), lambda i: (i, 0)),
        ],
        out_shape=[
            jax.ShapeDtypeStruct((N, H), jnp.float32),
            jax.ShapeDtypeStruct((N, 1), jnp.float32),
        ],
    )

    def staged_body(a_ref, g1_ref, dis_ref, b_ref, w_ref, h1_ref, g2_ref):
        dis = dis_ref[...]
        acc = a_ref[0] + a_ref[1] + g1_ref[...]
        h1 = jnp.maximum(acc * dis + b_ref[...], 0.0)
        h1_ref[...] = h1
        g2_ref[...] = jnp.dot(h1, w_ref[...], preferred_element_type=jnp.float32) * dis

    staged = pl.pallas_call(
        staged_body,
        grid=(NB,),
        in_specs=[
            pl.BlockSpec((_NCORE, BN, H), lambda i: (0, i, 0)),
            pl.BlockSpec((BN, H), lambda i: (i, 0)),
            pl.BlockSpec((BN, 1), lambda i: (i, 0)),
            pl.BlockSpec((1, H), lambda i: (0, 0)),
            pl.BlockSpec((H, H), lambda i: (0, 0)),
        ],
        out_specs=[
            pl.BlockSpec((BN, H), lambda i: (i, 0)),
            pl.BlockSpec((BN, H), lambda i: (i, 0)),
        ],
        out_shape=[
            jax.ShapeDtypeStruct((N, H), jnp.float32),
            jax.ShapeDtypeStruct((N, H), jnp.float32),
        ],
    )

    def stagef_body(a_ref, g2_ref, dis_ref, b_ref, h1_ref, batch_ref, nw_ref,
                    nb_ref, gw_ref, gb_ref, nl_ref, gl_ref, sums_ref, cnt_ref,
                    mx_ref):
        i = pl.program_id(0)
        dis = dis_ref[...]
        acc = a_ref[0] + a_ref[1] + g2_ref[...]
        h2 = jnp.maximum(acc * dis + b_ref[...], 0.0)
        h = jnp.concatenate([h1_ref[...], h2], axis=1)
        nl_ref[...] = (
            jnp.dot(h, nw_ref[...], preferred_element_type=jnp.float32) + nb_ref[...]
        )

        @pl.when(i == 0)
        def _():
            sums_ref[...] = jnp.zeros_like(sums_ref)
            cnt_ref[...] = jnp.zeros_like(cnt_ref)
            mx_ref[...] = jnp.full_like(mx_ref, -jnp.inf)

        brow = batch_ref[0]
        iota_g = lax.broadcasted_iota(jnp.int32, (G, BN), 0)
        maskf = (brow == iota_g).astype(jnp.float32)
        sums_ref[...] += jnp.dot(maskf, h, preferred_element_type=jnp.float32)
        cnt_ref[...] += jnp.sum(maskf, axis=1, keepdims=True)
        bcol = brow.reshape(BN, 1)
        for g in range(G):
            m = jnp.max(jnp.where(bcol == g, h, -jnp.inf), axis=0, keepdims=True)
            mx_ref[pl.ds(g, 1), :] = jnp.maximum(mx_ref[pl.ds(g, 1), :], m)

        cnt = cnt_ref[...]
        mean = sums_ref[...] / jnp.maximum(cnt, 1.0)
        mxp = jnp.where(cnt > 0, mx_ref[...], 0.0)
        pooled = jnp.concatenate([mean, mxp], axis=1)
        gl_ref[...] = (
            jnp.dot(pooled, gw_ref[...], preferred_element_type=jnp.float32)
            + gb_ref[...]
        )

    stagef = pl.pallas_call(
        stagef_body,
        grid=(NB,),
        in_specs=[
            pl.BlockSpec((_NCORE, BN, H), lambda i: (0, i, 0)),
            pl.BlockSpec((BN, H), lambda i: (i, 0)),
            pl.BlockSpec((BN, 1), lambda i: (i, 0)),
            pl.BlockSpec((1, H), lambda i: (0, 0)),
            pl.BlockSpec((BN, H), lambda i: (i, 0)),
            pl.BlockSpec((1, 1, BN), lambda i: (i, 0, 0)),
            pl.BlockSpec((H2, 1), lambda i: (0, 0)),
            pl.BlockSpec((1, 1), lambda i: (0, 0)),
            pl.BlockSpec((2 * H2, 1), lambda i: (0, 0)),
            pl.BlockSpec((1, 1), lambda i: (0, 0)),
        ],
        out_specs=[
            pl.BlockSpec((BN, 1), lambda i: (i, 0)),
            pl.BlockSpec((G, 1), lambda i: (0, 0)),
        ],
        out_shape=[
            jax.ShapeDtypeStruct((N, 1), jnp.float32),
            jax.ShapeDtypeStruct((G, 1), jnp.float32),
        ],
        scratch_shapes=[
            pltpu.VMEM((G, H2), jnp.float32),
            pltpu.VMEM((G, 1), jnp.float32),
            pltpu.VMEM((G, H2), jnp.float32),
        ],
        compiler_params=pltpu.CompilerParams(
            dimension_semantics=("arbitrary",),
        ),
    )

    return stageb, staged, stagef, (BN, NB)


def kernel(x, edge_index, batch, W1, b1, W2, b2, node_W, node_b, graph_W, graph_b):
    N, D = x.shape
    H = W1.shape[0]
    G = 16
    E = edge_index.shape[1]

    conv_sc, deg_sc, (epad, totrows, nrow) = _build_sc(E, N, H)
    stageb, staged, stagef, (BN, NB) = _build_tc(N, D, H, G)

    pad = epad - E
    pad_i = jnp.arange(pad, dtype=jnp.int32)
    src = jnp.concatenate(
        [edge_index[0], pad_i % N]).reshape(totrows, _M)
    dst = jnp.concatenate(
        [edge_index[1], N + pad_i % (nrow - N)]).reshape(totrows, _M)
    z_h = jnp.zeros((nrow, H), jnp.float32)
    z_16 = jnp.zeros((nrow, 16), jnp.float32)
    ones = jnp.ones((_M, 16), jnp.float32)

    degs = deg_sc(dst, z_16, ones)
    g1, dis = stageb(x, degs, W1.T)
    a1 = conv_sc(g1, src, dst, z_h)
    h1, g2 = staged(a1, g1, dis, b1.reshape(1, H), W2.T)
    a2 = conv_sc(g2, src, dst, z_h)
    nl, gl = stagef(
        a2, g2, dis, b2.reshape(1, H), h1,
        batch.reshape(NB, 1, BN),
        node_W.reshape(2 * H, 1), node_b.reshape(1, 1),
        graph_W.reshape(4 * H, 1), graph_b.reshape(1, 1),
    )
    return (gl.reshape(-1), nl.reshape(-1))


# speedup vs baseline: 32.2421x; 1.0264x over previous
import functools
import math

import jax
import jax.numpy as jnp
from jax import lax
from jax.experimental import pallas as pl
from jax.experimental.pallas import tpu as pltpu
from jax.experimental.pallas import tpu_sc as plsc

_NCORE = 2
_NSUB = 16
_M = 128
_CHUNK = 4


@functools.lru_cache(maxsize=None)
def _build_sc(E, N, H):
    ntiles = _NCORE * _NSUB
    rows_total = -(-E // _M)
    rows_per_tile = -(-(-(-rows_total // ntiles)) // _CHUNK) * _CHUNK
    totrows = ntiles * rows_per_tile
    epad = totrows * _M
    nchunk = rows_per_tile // _CHUNK
    nrow = -(-(N + 1) // 128) * 128
    init_rows = nrow // _NSUB

    mesh = plsc.VectorSubcoreMesh(core_axis_name="c", subcore_axis_name="s")
    sc_params = pltpu.CompilerParams(use_tc_tiling_on_sc=False)

    ch = 2
    nch = rows_per_tile // ch

    @functools.partial(
        pl.kernel,
        out_type=jax.ShapeDtypeStruct((_NCORE, nrow, H), jnp.float32),
        mesh=mesh,
        compiler_params=sc_params,
        scratch_types=[
            pltpu.VMEM((rows_per_tile, _M), jnp.int32),
            pltpu.VMEM((rows_per_tile, _M), jnp.int32),
            pltpu.VMEM((ch * _M, H), jnp.float32),
            pltpu.VMEM((ch * _M, H), jnp.float32),
            pltpu.VMEM_SHARED((nrow, H), jnp.float32),
            pltpu.SemaphoreType.DMA,
            pltpu.SemaphoreType.DMA,
        ],
    )
    def conv(g_hbm, src_hbm, dst_hbm, z_hbm, out_hbm, sidx, didx, buf0, buf1,
             acc, sem0, sem1):
        c = lax.axis_index("c")
        s = lax.axis_index("s")
        t = c * _NSUB + s
        ib = s * init_rows
        row_t = t * rows_per_tile
        pltpu.sync_copy(src_hbm.at[pl.ds(row_t, rows_per_tile)], sidx)
        pltpu.sync_copy(dst_hbm.at[pl.ds(row_t, rows_per_tile)], didx)
        pltpu.sync_copy(z_hbm.at[pl.ds(ib, init_rows)], acc.at[pl.ds(ib, init_rows)])
        plsc.subcore_barrier()

        bufs = (buf0, buf1)
        sems = (sem0, sem1)

        def fire(k, p):
            for j in range(ch):
                pltpu.async_copy(g_hbm.at[sidx.at[k * ch + j]],
                                 bufs[p].at[pl.ds(j * _M, _M)], sems[p])

        def wait(k, p):
            for j in range(ch):
                pltpu.make_async_copy(g_hbm.at[sidx.at[k * ch + j]],
                                      bufs[p].at[pl.ds(j * _M, _M)],
                                      sems[p]).wait()

        def scat(k, p):
            for j in range(ch):
                pltpu.sync_copy(bufs[p].at[pl.ds(j * _M, _M)],
                                acc.at[didx.at[k * ch + j]], add=True)

        fire(0, 0)

        @pl.loop(0, nch - 2, step=2)
        def _(k):
            fire(k + 1, 1)
            wait(k, 0)
            scat(k, 0)
            fire(k + 2, 0)
            wait(k + 1, 1)
            scat(k + 1, 1)

        fire(nch - 1, 1)
        wait(nch - 2, 0)
        scat(nch - 2, 0)
        wait(nch - 1, 1)
        scat(nch - 1, 1)

        plsc.subcore_barrier()
        pltpu.sync_copy(acc.at[pl.ds(ib, init_rows)], out_hbm.at[c, pl.ds(ib, init_rows)])

    @functools.partial(
        pl.kernel,
        out_type=jax.ShapeDtypeStruct((_NCORE, nrow, 16), jnp.float32),
        mesh=mesh,
        compiler_params=sc_params,
        scratch_types=[
            pltpu.VMEM((_CHUNK, _M), jnp.int32),
            pltpu.VMEM((_M, 16), jnp.float32),
            pltpu.VMEM_SHARED((nrow, 16), jnp.float32),
        ],
    )
    def deg(dst_hbm, z_hbm, ones_hbm, out_hbm, didx, ones, dacc):
        c = lax.axis_index("c")
        s = lax.axis_index("s")
        t = c * _NSUB + s
        pltpu.sync_copy(ones_hbm, ones)
        ib = s * init_rows
        pltpu.sync_copy(z_hbm.at[pl.ds(ib, init_rows)], dacc.at[pl.ds(ib, init_rows)])
        plsc.subcore_barrier()

        @pl.loop(0, nchunk)
        def _(k):
            row0 = t * rows_per_tile + k * _CHUNK
            pltpu.sync_copy(dst_hbm.at[pl.ds(row0, _CHUNK)], didx)
            for j in range(_CHUNK):
                pltpu.sync_copy(ones, dacc.at[didx.at[j]], add=True)

        plsc.subcore_barrier()
        pltpu.sync_copy(dacc.at[pl.ds(ib, init_rows)], out_hbm.at[c, pl.ds(ib, init_rows)])

    return conv, deg, (epad, totrows, nrow)


@functools.lru_cache(maxsize=None)
def _build_tc(N, D, H, G):
    BN = 1000 if N % 1000 == 0 else N
    NB = N // BN
    H2 = 2 * H

    def stageb_body(x_ref, deg_ref, w_ref, g_ref, dis_ref):
        d = deg_ref[0] + deg_ref[1]
        dis = lax.rsqrt(d[:, 0:1] + 1.0)
        h = jnp.dot(x_ref[...], w_ref[...], preferred_element_type=jnp.float32)
        g_ref[...] = h * dis
        dis_ref[...] = dis

    stageb = pl.pallas_call(
        stageb_body,
        grid=(NB,),
        in_specs=[
            pl.BlockSpec((BN, D), lambda i: (i, 0)),
            pl.BlockSpec((_NCORE, BN, 16), lambda i: (0, i, 0)),
            pl.BlockSpec((D, H), lambda i: (0, 0)),
        ],
        out_specs=[
            pl.BlockSpec((BN, H), lambda i: (i, 0)),
            pl.BlockSpec((BN, 1), lambda i: (i, 0)),
        ],
        out_shape=[
            jax.ShapeDtypeStruct((N, H), jnp.float32),
            jax.ShapeDtypeStruct((N, 1), jnp.float32),
        ],
    )

    def staged_body(a_ref, g1_ref, dis_ref, b_ref, w_ref, h1_ref, g2_ref):
        dis = dis_ref[...]
        acc = a_ref[0] + a_ref[1] + g1_ref[...]
        h1 = jnp.maximum(acc * dis + b_ref[...], 0.0)
        h1_ref[...] = h1
        g2_ref[...] = jnp.dot(h1, w_ref[...], preferred_element_type=jnp.float32) * dis

    staged = pl.pallas_call(
        staged_body,
        grid=(NB,),
        in_specs=[
            pl.BlockSpec((_NCORE, BN, H), lambda i: (0, i, 0)),
            pl.BlockSpec((BN, H), lambda i: (i, 0)),
            pl.BlockSpec((BN, 1), lambda i: (i, 0)),
            pl.BlockSpec((1, H), lambda i: (0, 0)),
            pl.BlockSpec((H, H), lambda i: (0, 0)),
        ],
        out_specs=[
            pl.BlockSpec((BN, H), lambda i: (i, 0)),
            pl.BlockSpec((BN, H), lambda i: (i, 0)),
        ],
        out_shape=[
            jax.ShapeDtypeStruct((N, H), jnp.float32),
            jax.ShapeDtypeStruct((N, H), jnp.float32),
        ],
    )

    def stagef_body(a_ref, g2_ref, dis_ref, b_ref, h1_ref, batch_ref, nw_ref,
                    nb_ref, gw_ref, gb_ref, nl_ref, gl_ref, sums_ref, cnt_ref,
                    mx_ref):
        i = pl.program_id(0)
        dis = dis_ref[...]
        acc = a_ref[0] + a_ref[1] + g2_ref[...]
        h2 = jnp.maximum(acc * dis + b_ref[...], 0.0)
        h = jnp.concatenate([h1_ref[...], h2], axis=1)
        nl_ref[...] = (
            jnp.dot(h, nw_ref[...], preferred_element_type=jnp.float32) + nb_ref[...]
        )

        @pl.when(i == 0)
        def _():
            sums_ref[...] = jnp.zeros_like(sums_ref)
            cnt_ref[...] = jnp.zeros_like(cnt_ref)
            mx_ref[...] = jnp.full_like(mx_ref, -jnp.inf)

        brow = batch_ref[0]
        iota_g = lax.broadcasted_iota(jnp.int32, (G, BN), 0)
        maskf = (brow == iota_g).astype(jnp.float32)
        sums_ref[...] += jnp.dot(maskf, h, preferred_element_type=jnp.float32)
        cnt_ref[...] += jnp.sum(maskf, axis=1, keepdims=True)
        bcol = brow.reshape(BN, 1)
        b_lo = brow[0, 0]
        b_hi = brow[0, BN - 1]
        for g in range(G):
            @pl.when(jnp.logical_and(g >= b_lo, g <= b_hi))
            def _():
                m = jnp.max(jnp.where(bcol == g, h, -jnp.inf), axis=0,
                            keepdims=True)
                mx_ref[pl.ds(g, 1), :] = jnp.maximum(mx_ref[pl.ds(g, 1), :], m)

        cnt = cnt_ref[...]
        mean = sums_ref[...] / jnp.maximum(cnt, 1.0)
        mxp = jnp.where(cnt > 0, mx_ref[...], 0.0)
        pooled = jnp.concatenate([mean, mxp], axis=1)
        gl_ref[...] = (
            jnp.dot(pooled, gw_ref[...], preferred_element_type=jnp.float32)
            + gb_ref[...]
        )

    stagef = pl.pallas_call(
        stagef_body,
        grid=(NB,),
        in_specs=[
            pl.BlockSpec((_NCORE, BN, H), lambda i: (0, i, 0)),
            pl.BlockSpec((BN, H), lambda i: (i, 0)),
            pl.BlockSpec((BN, 1), lambda i: (i, 0)),
            pl.BlockSpec((1, H), lambda i: (0, 0)),
            pl.BlockSpec((BN, H), lambda i: (i, 0)),
            pl.BlockSpec((1, 1, BN), lambda i: (i, 0, 0)),
            pl.BlockSpec((H2, 1), lambda i: (0, 0)),
            pl.BlockSpec((1, 1), lambda i: (0, 0)),
            pl.BlockSpec((2 * H2, 1), lambda i: (0, 0)),
            pl.BlockSpec((1, 1), lambda i: (0, 0)),
        ],
        out_specs=[
            pl.BlockSpec((BN, 1), lambda i: (i, 0)),
            pl.BlockSpec((G, 1), lambda i: (0, 0)),
        ],
        out_shape=[
            jax.ShapeDtypeStruct((N, 1), jnp.float32),
            jax.ShapeDtypeStruct((G, 1), jnp.float32),
        ],
        scratch_shapes=[
            pltpu.VMEM((G, H2), jnp.float32),
            pltpu.VMEM((G, 1), jnp.float32),
            pltpu.VMEM((G, H2), jnp.float32),
        ],
        compiler_params=pltpu.CompilerParams(
            dimension_semantics=("arbitrary",),
        ),
    )

    return stageb, staged, stagef, (BN, NB)


def kernel(x, edge_index, batch, W1, b1, W2, b2, node_W, node_b, graph_W, graph_b):
    N, D = x.shape
    H = W1.shape[0]
    G = 16
    E = edge_index.shape[1]

    conv_sc, deg_sc, (epad, totrows, nrow) = _build_sc(E, N, H)
    stageb, staged, stagef, (BN, NB) = _build_tc(N, D, H, G)

    pad = epad - E
    pad_i = jnp.arange(pad, dtype=jnp.int32)
    src = jnp.concatenate(
        [edge_index[0], pad_i % N]).reshape(totrows, _M)
    dst = jnp.concatenate(
        [edge_index[1], N + pad_i % (nrow - N)]).reshape(totrows, _M)
    z_h = jnp.zeros((nrow, H), jnp.float32)
    z_16 = jnp.zeros((nrow, 16), jnp.float32)
    ones = jnp.ones((_M, 16), jnp.float32)

    degs = deg_sc(dst, z_16, ones)
    g1, dis = stageb(x, degs, W1.T)
    a1 = conv_sc(g1, src, dst, z_h)
    h1, g2 = staged(a1, g1, dis, b1.reshape(1, H), W2.T)
    a2 = conv_sc(g2, src, dst, z_h)
    nl, gl = stagef(
        a2, g2, dis, b2.reshape(1, H), h1,
        batch.reshape(NB, 1, BN),
        node_W.reshape(2 * H, 1), node_b.reshape(1, 1),
        graph_W.reshape(4 * H, 1), graph_b.reshape(1, 1),
    )
    return (gl.reshape(-1), nl.reshape(-1))
